# baseline (speedup 1.0000x reference)
import functools
import math

import jax
import jax.numpy as jnp
from jax import lax
from jax.experimental import pallas as pl
from jax.experimental.pallas import tpu as pltpu
from jax.experimental.pallas import tpu_sc as plsc

VOCAB_SIZE = 1000000
D = 64
DP = 128
SCALE = math.sqrt(D)
NC, NS = 2, 16
NW = NC * NS
B = 4096 * 200
PER_W = B // NW
C = 160
NCH = PER_W // C
L = 16


VB = 32768


def _transpose_body(tt_ref, out_ref):
    out_ref[:, :D] = tt_ref[...].T


_transpose_pad = pl.pallas_call(
    _transpose_body,
    grid=((VOCAB_SIZE + VB - 1) // VB,),
    in_specs=[pl.BlockSpec((D, VB), lambda i: (0, i))],
    out_specs=pl.BlockSpec((VB, DP), lambda i: (i, 0)),
    out_shape=jax.ShapeDtypeStruct((VOCAB_SIZE, DP), jnp.float32),
)


def _make_sc_kernel():
    mesh = plsc.VectorSubcoreMesh(core_axis_name="c", subcore_axis_name="s")

    @functools.partial(
        pl.kernel,
        out_type=jax.ShapeDtypeStruct((B, D), jnp.float32),
        mesh=mesh,
        compiler_params=pltpu.CompilerParams(use_tc_tiling_on_sc=True),
        scratch_types=[
            pltpu.VMEM((PER_W,), jnp.int32),
            pltpu.VMEM((C, DP), jnp.float32),
            pltpu.VMEM((C, DP), jnp.float32),
            pltpu.VMEM((C, D), jnp.float32),
            pltpu.VMEM((C, D), jnp.float32),
            pltpu.SemaphoreType.DMA,
            pltpu.SemaphoreType.DMA,
            pltpu.SemaphoreType.DMA,
            pltpu.SemaphoreType.DMA,
        ],
    )
    def emb(x_hbm, tab_hbm, out_hbm, idx_v, buf0, buf1, cmp0, cmp1,
            sem0, sem1, osem0, osem1):
        wid = lax.axis_index("s") * NC + lax.axis_index("c")
        base = wid * PER_W
        pltpu.sync_copy(x_hbm.at[pl.ds(base, PER_W)], idx_v)
        bufs = (buf0, buf1)
        cmps = (cmp0, cmp1)
        sems = (sem0, sem1)
        osems = (osem0, osem1)

        def issue(ch, b):
            pltpu.async_copy(
                tab_hbm.at[idx_v.at[pl.ds(ch * C, C)]], bufs[b], sems[b])

        def wait_gather(ch, b):
            pltpu.make_async_copy(
                tab_hbm.at[idx_v.at[pl.ds(ch * C, C)]], bufs[b],
                sems[b]).wait()

        for b in range(2):
            issue(b, b)

        @pl.loop(0, NCH, step=2)
        def _chunk(ch):
            for b in range(2):
                cc = ch + b
                wait_gather(cc, b)
                buf, cmp = bufs[b], cmps[b]

                @pl.when(cc >= 2)
                def _drain():
                    pltpu.make_async_copy(
                        cmp, out_hbm.at[pl.ds(base + (cc - 2) * C, C)],
                        osems[b]).wait()

                @pl.loop(0, C, unroll=8)
                def _row(k):
                    for c in range(0, D, L):
                        cmp[k, pl.ds(c, L)] = buf[k, pl.ds(c, L)] * SCALE

                pltpu.async_copy(
                    cmp, out_hbm.at[pl.ds(base + cc * C, C)], osems[b])

                @pl.when(cc + 2 < NCH)
                def _next():
                    issue(cc + 2, b)

        for b in range(2):
            pltpu.make_async_copy(
                cmps[b],
                out_hbm.at[pl.ds(base + (NCH - 2 + b) * C, C)],
                osems[b]).wait()

    return emb


_emb = _make_sc_kernel()


def kernel(x, table):
    xf = x.astype(jnp.int32).reshape(-1)
    tab2 = _transpose_pad(table.T)
    out = _emb(xf, tab2)
    return out.reshape(x.shape[0], x.shape[1], D)


# speedup vs baseline: 1.2458x; 1.2458x over previous
import functools
import math

import jax
import jax.numpy as jnp
from jax import lax
from jax.experimental import pallas as pl
from jax.experimental.pallas import tpu as pltpu
from jax.experimental.pallas import tpu_sc as plsc

VOCAB_SIZE = 1000000
D = 64
DP = 128
SCALE = math.sqrt(D)
NC, NS = 2, 16
NW = NC * NS
B = 4096 * 200
PER_W = B // NW
C = 160
NCH = PER_W // C
L = 16


VB = 32768


def _transpose_body(tt_ref, out_ref):
    out_ref[:, :D] = tt_ref[...].T


_transpose_pad = pl.pallas_call(
    _transpose_body,
    grid=((VOCAB_SIZE + VB - 1) // VB,),
    in_specs=[pl.BlockSpec((D, VB), lambda i: (0, i))],
    out_specs=pl.BlockSpec((VB, DP), lambda i: (i, 0)),
    out_shape=jax.ShapeDtypeStruct((VOCAB_SIZE, DP), jnp.float32),
)


def _make_sc_kernel():
    mesh = plsc.VectorSubcoreMesh(core_axis_name="c", subcore_axis_name="s")

    @functools.partial(
        pl.kernel,
        out_type=jax.ShapeDtypeStruct((B, D), jnp.float32),
        mesh=mesh,
        compiler_params=pltpu.CompilerParams(use_tc_tiling_on_sc=True),
        scratch_types=[
            pltpu.VMEM((PER_W,), jnp.int32),
            pltpu.VMEM((C, DP), jnp.float32),
            pltpu.VMEM((C, DP), jnp.float32),
            pltpu.VMEM((C, D), jnp.float32),
            pltpu.VMEM((C, D), jnp.float32),
            pltpu.SemaphoreType.DMA,
            pltpu.SemaphoreType.DMA,
            pltpu.SemaphoreType.DMA,
            pltpu.SemaphoreType.DMA,
        ],
    )
    def emb(x_hbm, tab_hbm, out_hbm, idx_v, buf0, buf1, cmp0, cmp1,
            sem0, sem1, osem0, osem1):
        wid = lax.axis_index("s") * NC + lax.axis_index("c")
        base = wid * PER_W
        pltpu.sync_copy(x_hbm.at[pl.ds(base, PER_W)], idx_v)
        bufs = (buf0, buf1)
        cmps = (cmp0, cmp1)
        sems = (sem0, sem1)
        osems = (osem0, osem1)

        def issue(ch, b):
            pltpu.async_copy(
                tab_hbm.at[idx_v.at[pl.ds(ch * C, C)]], bufs[b], sems[b])

        def wait_gather(ch, b):
            pltpu.make_async_copy(
                tab_hbm.at[idx_v.at[pl.ds(ch * C, C)]], bufs[b],
                sems[b]).wait()

        for b in range(2):
            issue(b, b)

        @pl.loop(0, NCH, step=2)
        def _chunk(ch):
            for b in range(2):
                cc = ch + b
                wait_gather(cc, b)
                buf, cmp = bufs[b], cmps[b]

                @pl.when(cc >= 2)
                def _drain():
                    pltpu.make_async_copy(
                        cmp, out_hbm.at[pl.ds(base + (cc - 2) * C, C)],
                        osems[b]).wait()

                @pl.loop(0, C)
                def _row(k):
                    for c in range(0, D, L):
                        cmp[k, pl.ds(c, L)] = buf[k, pl.ds(c, L)] * SCALE

                pltpu.async_copy(
                    cmp, out_hbm.at[pl.ds(base + cc * C, C)], osems[b])

                @pl.when(cc + 2 < NCH)
                def _next():
                    issue(cc + 2, b)

        for b in range(2):
            pltpu.make_async_copy(
                cmps[b],
                out_hbm.at[pl.ds(base + (NCH - 2 + b) * C, C)],
                osems[b]).wait()

    return emb


_emb = _make_sc_kernel()


def kernel(x, table):
    xf = x.astype(jnp.int32).reshape(-1)
    tab2 = _transpose_pad(table.T)
    out = _emb(xf, tab2)
    return out.reshape(x.shape[0], x.shape[1], D)
